# Initial kernel scaffold; baseline (speedup 1.0000x reference)
import jax
import jax.numpy as jnp
from jax.experimental import pallas as pl


def kernel(h, vn_h, segment_ids, W, b):
    raise NotImplementedError("write your pallas kernel here")


# speedup vs baseline: 1.3402x; 1.3402x over previous
import functools

import jax
import jax.numpy as jnp
from jax import lax
from jax.experimental import pallas as pl
from jax.experimental.pallas import tpu as pltpu
from jax.experimental.pallas import tpu_sc as plsc

N = 100000
B = 512
DIM = 128

NC = 2
NS = 16
NW = NC * NS

BLK = 200
NBLOCKS = N // BLK
NBLK_MAX = -(-NBLOCKS // NW)
EXTRA = NBLOCKS - NW * (NBLK_MAX - 1)
LANES = 16
SUB = DIM // LANES


def _worker_id():
    return lax.axis_index("s") * NC + lax.axis_index("c")


def _worker_blocks(wid):
    nblk = jnp.where(wid < EXTRA, NBLK_MAX, NBLK_MAX - 1)
    blk0 = (NBLK_MAX - 1) * wid + jnp.minimum(wid, EXTRA)
    return blk0, nblk


def _pool_kernel(h_hbm, seg_hbm, out_hbm, pool_sh, hblk, idx, zblk):
    cid = lax.axis_index("c")
    sid = lax.axis_index("s")
    wid = _worker_id()
    blk0, nblk = _worker_blocks(wid)

    rows_per_tile = B // NS
    for r in range(rows_per_tile):
        for t in range(SUB):
            zblk[r, pl.ds(t * LANES, LANES)] = jnp.zeros((LANES,), jnp.float32)
    pltpu.sync_copy(zblk, pool_sh.at[pl.ds(sid * rows_per_tile, rows_per_tile)])
    plsc.subcore_barrier()

    @pl.loop(0, NBLK_MAX)
    def _(j):
        @pl.when(j < nblk)
        def _():
            base = (blk0 + j) * BLK
            pltpu.sync_copy(seg_hbm.at[pl.ds(base, BLK)], idx)
            pltpu.sync_copy(h_hbm.at[pl.ds(base, BLK)], hblk)
            pltpu.sync_copy(hblk, pool_sh.at[idx], add=True)

    plsc.subcore_barrier()
    pltpu.sync_copy(
        pool_sh.at[pl.ds(sid * rows_per_tile, rows_per_tile)],
        out_hbm.at[cid, pl.ds(sid * rows_per_tile, rows_per_tile)],
    )


def _sc_pool(h, seg):
    mesh = plsc.VectorSubcoreMesh(core_axis_name="c", subcore_axis_name="s")
    kern = functools.partial(
        pl.kernel,
        mesh=mesh,
        out_type=jax.ShapeDtypeStruct((NC, B, DIM), jnp.float32),
        scratch_types=[
            pltpu.VMEM_SHARED((B, DIM), jnp.float32),
            pltpu.VMEM((BLK, DIM), jnp.float32),
            pltpu.VMEM((BLK,), jnp.int32),
            pltpu.VMEM((B // NS, DIM), jnp.float32),
        ],
    )(_pool_kernel)
    return kern(h, seg)


def _bcast_kernel(h_hbm, seg_hbm, vn_hbm, out_hbm, hblk, gblk, idx, sem):
    wid = _worker_id()
    blk0, nblk = _worker_blocks(wid)

    @pl.loop(0, NBLK_MAX)
    def _(j):
        @pl.when(j < nblk)
        def _():
            base = (blk0 + j) * BLK
            pltpu.sync_copy(seg_hbm.at[pl.ds(base, BLK)], idx)
            pltpu.sync_copy(h_hbm.at[pl.ds(base, BLK)], hblk)
            pltpu.async_copy(vn_hbm.at[idx], gblk, sem).wait()

            @pl.loop(0, BLK)
            def _(r):
                for t in range(SUB):
                    sl = pl.ds(t * LANES, LANES)
                    hblk[r, sl] = hblk[r, sl] + gblk[r, sl]

            pltpu.sync_copy(hblk, out_hbm.at[pl.ds(base, BLK)])


def _sc_broadcast(h, seg, vn_new):
    mesh = plsc.VectorSubcoreMesh(core_axis_name="c", subcore_axis_name="s")
    kern = functools.partial(
        pl.kernel,
        mesh=mesh,
        out_type=jax.ShapeDtypeStruct((N, DIM), jnp.float32),
        scratch_types=[
            pltpu.VMEM((BLK, DIM), jnp.float32),
            pltpu.VMEM((BLK, DIM), jnp.float32),
            pltpu.VMEM((BLK,), jnp.int32),
            pltpu.SemaphoreType.DMA,
        ],
    )(_bcast_kernel)
    return kern(h, seg, vn_new)


def _fc_body(pools_ref, vn_ref, w_ref, b_ref, out_ref):
    pool = pools_ref[0] + pools_ref[1]
    x = vn_ref[...] + pool
    y = jnp.dot(x, w_ref[...], preferred_element_type=jnp.float32) + b_ref[...]
    out_ref[...] = vn_ref[...] + jnp.maximum(y, 0.0)


def _tc_fc(pools, vn_h, W, b):
    return pl.pallas_call(
        _fc_body,
        out_shape=jax.ShapeDtypeStruct((B, DIM), jnp.float32),
    )(pools, vn_h, W, b.reshape(1, DIM))


def kernel(h, vn_h, segment_ids, W, b):
    seg = segment_ids.astype(jnp.int32)
    pools = _sc_pool(h, seg)
    vn_new = _tc_fc(pools, vn_h, W, b)
    h_new = _sc_broadcast(h, seg, vn_new)
    return (vn_new, h_new)


# speedup vs baseline: 2.1188x; 1.5809x over previous
import functools

import jax
import jax.numpy as jnp
from jax import lax
from jax.experimental import pallas as pl
from jax.experimental.pallas import tpu as pltpu
from jax.experimental.pallas import tpu_sc as plsc

N = 100000
B = 512
DIM = 128

NC = 2
NS = 16
NW = NC * NS

BLK = 400
NBLOCKS = N // BLK
NBLK_MAX = -(-NBLOCKS // NW)
EXTRA = NBLOCKS - NW * (NBLK_MAX - 1)
LANES = 16
SUB = DIM // LANES


def _worker_id():
    return lax.axis_index("s") * NC + lax.axis_index("c")


def _worker_blocks(wid):
    nblk = jnp.where(wid < EXTRA, NBLK_MAX, NBLK_MAX - 1)
    blk0 = (NBLK_MAX - 1) * wid + jnp.minimum(wid, EXTRA)
    return blk0, nblk


def _pool_kernel(h_hbm, seg_hbm, out_hbm, pool_sh, hblk, idx, zblk):
    cid = lax.axis_index("c")
    sid = lax.axis_index("s")
    wid = _worker_id()
    blk0, nblk = _worker_blocks(wid)

    rows_per_tile = B // NS
    for r in range(rows_per_tile):
        for t in range(SUB):
            zblk[r, pl.ds(t * LANES, LANES)] = jnp.zeros((LANES,), jnp.float32)
    pltpu.sync_copy(zblk, pool_sh.at[pl.ds(sid * rows_per_tile, rows_per_tile)])
    plsc.subcore_barrier()

    @pl.loop(0, NBLK_MAX)
    def _(j):
        @pl.when(j < nblk)
        def _():
            base = (blk0 + j) * BLK
            pltpu.sync_copy(seg_hbm.at[pl.ds(base, BLK)], idx)
            pltpu.sync_copy(h_hbm.at[pl.ds(base, BLK)], hblk)
            pltpu.sync_copy(hblk, pool_sh.at[idx], add=True)

    plsc.subcore_barrier()
    pltpu.sync_copy(
        pool_sh.at[pl.ds(sid * rows_per_tile, rows_per_tile)],
        out_hbm.at[cid, pl.ds(sid * rows_per_tile, rows_per_tile)],
    )


def _sc_pool(h, seg):
    mesh = plsc.VectorSubcoreMesh(core_axis_name="c", subcore_axis_name="s")
    kern = functools.partial(
        pl.kernel,
        mesh=mesh,
        out_type=jax.ShapeDtypeStruct((NC, B, DIM), jnp.float32),
        scratch_types=[
            pltpu.VMEM_SHARED((B, DIM), jnp.float32),
            pltpu.VMEM((BLK, DIM), jnp.float32),
            pltpu.VMEM((BLK,), jnp.int32),
            pltpu.VMEM((B // NS, DIM), jnp.float32),
        ],
    )(_pool_kernel)
    return kern(h, seg)


def _bcast_kernel(h_hbm, seg_hbm, vn_hbm, out_hbm, sp, gblk, idx, iota, sem):
    sid = lax.axis_index("s")
    wid = _worker_id()
    blk0, nblk = _worker_blocks(wid)

    for t in range(BLK // LANES):
        iota[pl.ds(t * LANES, LANES)] = lax.iota(jnp.int32, LANES) + t * LANES
    hsp = sp.at[sid]

    @pl.loop(0, NBLK_MAX)
    def _(j):
        @pl.when(j < nblk)
        def _():
            base = (blk0 + j) * BLK
            pltpu.sync_copy(seg_hbm.at[pl.ds(base, BLK)], idx)
            gather = pltpu.async_copy(vn_hbm.at[idx], gblk, sem)
            pltpu.sync_copy(h_hbm.at[pl.ds(base, BLK)], hsp)
            gather.wait()
            pltpu.sync_copy(gblk, hsp.at[iota], add=True)
            pltpu.sync_copy(hsp, out_hbm.at[pl.ds(base, BLK)])


def _sc_broadcast(h, seg, vn_new):
    mesh = plsc.VectorSubcoreMesh(core_axis_name="c", subcore_axis_name="s")
    kern = functools.partial(
        pl.kernel,
        mesh=mesh,
        out_type=jax.ShapeDtypeStruct((N, DIM), jnp.float32),
        scratch_types=[
            pltpu.VMEM_SHARED((NS, BLK, DIM), jnp.float32),
            pltpu.VMEM((BLK, DIM), jnp.float32),
            pltpu.VMEM((BLK,), jnp.int32),
            pltpu.VMEM((BLK,), jnp.int32),
            pltpu.SemaphoreType.DMA,
        ],
    )(_bcast_kernel)
    return kern(h, seg, vn_new)


def _fc_body(pools_ref, vn_ref, w_ref, b_ref, out_ref):
    pool = pools_ref[0] + pools_ref[1]
    x = vn_ref[...] + pool
    y = jnp.dot(x, w_ref[...], preferred_element_type=jnp.float32) + b_ref[...]
    out_ref[...] = vn_ref[...] + jnp.maximum(y, 0.0)


def _tc_fc(pools, vn_h, W, b):
    return pl.pallas_call(
        _fc_body,
        out_shape=jax.ShapeDtypeStruct((B, DIM), jnp.float32),
    )(pools, vn_h, W, b.reshape(1, DIM))


def kernel(h, vn_h, segment_ids, W, b):
    seg = segment_ids.astype(jnp.int32)
    pools = _sc_pool(h, seg)
    vn_new = _tc_fc(pools, vn_h, W, b)
    h_new = _sc_broadcast(h, seg, vn_new)
    return (vn_new, h_new)


# speedup vs baseline: 3.7753x; 1.7818x over previous
import functools

import jax
import jax.numpy as jnp
from jax import lax
from jax.experimental import pallas as pl
from jax.experimental.pallas import tpu as pltpu
from jax.experimental.pallas import tpu_sc as plsc

N = 100000
B = 512
DIM = 128

NC = 2
NS = 16
NW = NC * NS

LANES = 16
SUB = DIM // LANES

BLK = 400
NBLOCKS = N // BLK
NBLK_MAX = -(-NBLOCKS // NW)
EXTRA = NBLOCKS - NW * (NBLK_MAX - 1)

RB = 2000
NTB = N // RB


def _worker_id():
    return lax.axis_index("s") * NC + lax.axis_index("c")


def _worker_blocks(wid):
    nblk = jnp.where(wid < EXTRA, NBLK_MAX, NBLK_MAX - 1)
    blk0 = (NBLK_MAX - 1) * wid + jnp.minimum(wid, EXTRA)
    return blk0, nblk


def _pool_kernel(h_hbm, seg_hbm, out_hbm, pool_sh, hblk, idx, zblk):
    cid = lax.axis_index("c")
    sid = lax.axis_index("s")
    wid = _worker_id()
    blk0, nblk = _worker_blocks(wid)

    rows_per_tile = B // NS
    for r in range(rows_per_tile):
        for t in range(SUB):
            zblk[r, pl.ds(t * LANES, LANES)] = jnp.zeros((LANES,), jnp.float32)
    pltpu.sync_copy(zblk, pool_sh.at[pl.ds(sid * rows_per_tile, rows_per_tile)])
    plsc.subcore_barrier()

    @pl.loop(0, NBLK_MAX)
    def _(j):
        @pl.when(j < nblk)
        def _():
            base = (blk0 + j) * BLK
            pltpu.sync_copy(seg_hbm.at[pl.ds(base, BLK)], idx)
            pltpu.sync_copy(h_hbm.at[pl.ds(base, BLK)], hblk)
            pltpu.sync_copy(hblk, pool_sh.at[idx], add=True)

    plsc.subcore_barrier()
    pltpu.sync_copy(
        pool_sh.at[pl.ds(sid * rows_per_tile, rows_per_tile)],
        out_hbm.at[cid, pl.ds(sid * rows_per_tile, rows_per_tile)],
    )


def _sc_pool(h, seg):
    mesh = plsc.VectorSubcoreMesh(core_axis_name="c", subcore_axis_name="s")
    kern = functools.partial(
        pl.kernel,
        mesh=mesh,
        out_type=jax.ShapeDtypeStruct((NC, B, DIM), jnp.float32),
        scratch_types=[
            pltpu.VMEM_SHARED((B, DIM), jnp.float32),
            pltpu.VMEM((BLK, DIM), jnp.float32),
            pltpu.VMEM((BLK,), jnp.int32),
            pltpu.VMEM((B // NS, DIM), jnp.float32),
        ],
    )(_pool_kernel)
    return kern(h, seg)


def _fc_body(pools_ref, vn_ref, w_ref, b_ref, out_ref, hi_ref, lo_ref):
    pool = pools_ref[0] + pools_ref[1]
    x = vn_ref[...] + pool
    y = jnp.dot(x, w_ref[...], preferred_element_type=jnp.float32) + b_ref[...]
    vn_new = vn_ref[...] + jnp.maximum(y, 0.0)
    out_ref[...] = vn_new
    hi = vn_new.astype(jnp.bfloat16)
    hi_ref[...] = hi
    lo_ref[...] = (vn_new - hi.astype(jnp.float32)).astype(jnp.bfloat16)


def _tc_fc(pools, vn_h, W, b):
    return pl.pallas_call(
        _fc_body,
        out_shape=(
            jax.ShapeDtypeStruct((B, DIM), jnp.float32),
            jax.ShapeDtypeStruct((B, DIM), jnp.bfloat16),
            jax.ShapeDtypeStruct((B, DIM), jnp.bfloat16),
        ),
    )(pools, vn_h, W, b.reshape(1, DIM))


def _tc_bcast_body(seg_ref, h_ref, vhi_ref, vlo_ref, out_ref):
    seg = seg_ref[...]
    ids = lax.broadcasted_iota(jnp.int32, (RB, B), 1)
    onehot = (seg == ids).astype(jnp.bfloat16)
    acc = jnp.dot(onehot, vhi_ref[...], preferred_element_type=jnp.float32)
    acc = acc + jnp.dot(onehot, vlo_ref[...], preferred_element_type=jnp.float32)
    out_ref[...] = h_ref[...] + acc


def _tc_broadcast(h, seg, vn_hi, vn_lo):
    seg_col = seg.reshape(N, 1)
    return pl.pallas_call(
        _tc_bcast_body,
        grid=(NTB,),
        in_specs=[
            pl.BlockSpec((RB, 1), lambda i: (i, 0)),
            pl.BlockSpec((RB, DIM), lambda i: (i, 0)),
            pl.BlockSpec((B, DIM), lambda i: (0, 0)),
            pl.BlockSpec((B, DIM), lambda i: (0, 0)),
        ],
        out_specs=pl.BlockSpec((RB, DIM), lambda i: (i, 0)),
        out_shape=jax.ShapeDtypeStruct((N, DIM), jnp.float32),
    )(seg_col, h, vn_hi, vn_lo)


def kernel(h, vn_h, segment_ids, W, b):
    seg = segment_ids.astype(jnp.int32)
    pools = _sc_pool(h, seg)
    vn_new, vn_hi, vn_lo = _tc_fc(pools, vn_h, W, b)
    h_new = _tc_broadcast(h, seg, vn_hi, vn_lo)
    return (vn_new, h_new)
